# baseline (speedup 1.0000x reference)
import functools

import jax
import jax.numpy as jnp
from jax import lax
from jax.experimental import pallas as pl
from jax.experimental.pallas import tpu as pltpu
from jax.experimental.pallas import tpu_sc as plsc

L = 16
NC, NS = 2, 16
NW = NC * NS
CHUNK = 128
GROWS = 4


def _fm_call(x_flat, emb_table, fc_flat, bias16, B, F, D, total):
    field_size = total // F
    rpw = B // NW
    nch = rpw // CHUNK
    ppc = CHUNK * F
    gb = GROWS * F
    ngath = ppc // gb
    ng = ppc // L

    mesh = plsc.VectorSubcoreMesh(
        core_axis_name="c", subcore_axis_name="s", num_cores=NC, num_subcores=NS)

    @functools.partial(
        pl.kernel,
        out_type=jax.ShapeDtypeStruct((B,), jnp.float32),
        mesh=mesh,
        scratch_types=[
            pltpu.VMEM((ppc,), jnp.int32),
            pltpu.VMEM((ppc,), jnp.int32),
            pltpu.VMEM((ppc,), jnp.int32),
            pltpu.VMEM((ppc, D), jnp.float32),
            pltpu.VMEM((ppc,), jnp.float32),
            pltpu.VMEM((rpw,), jnp.float32),
            pltpu.VMEM((L,), jnp.float32),
            pltpu.SemaphoreType.DMA,
        ],
        compiler_params=pltpu.CompilerParams(
            needs_layout_passes=False, use_tc_tiling_on_sc=False),
    )
    def fm(emb_hbm, fc_hbm, x_hbm, b_hbm, out_hbm,
           xbuf, idxbuf, fidxbuf, ebuf, fcbuf, obuf, bbuf, sem):
        w = lax.axis_index("s") * NC + lax.axis_index("c")
        pltpu.sync_copy(b_hbm, bbuf)
        iota = lax.iota(jnp.int32, L)
        m_tail = jnp.where(iota < (F - L), 1.0, 0.0)
        m_last = iota == (L - 1)
        zero_i = iota * 0
        bias_v = bbuf[pl.ds(0, L)]

        def chunk_body(c, _):
            base_pair = w * (rpw * F) + c * ppc
            pltpu.sync_copy(x_hbm.at[pl.ds(base_pair, ppc)], xbuf)

            def idx_body(g, _):
                off = pl.multiple_of(g * L, L)
                xv = xbuf[pl.ds(off, L)]
                fv = (iota + g * L) % F
                row = xv + fv * field_size
                idxbuf[pl.ds(off, L)] = row * 8
                fidxbuf[pl.ds(off, L)] = row * 128
                return 0

            lax.fori_loop(0, ng, idx_body, 0)

            copies = []
            for j in range(ngath):
                copies.append(pltpu.async_copy(
                    emb_hbm.at[idxbuf.at[pl.ds(j * gb, gb)]],
                    ebuf.at[pl.ds(j * gb, gb)], sem))
                copies.append(pltpu.async_copy(
                    fc_hbm.at[fidxbuf.at[pl.ds(j * gb, gb)]],
                    fcbuf.at[pl.ds(j * gb, gb)], sem))
            for cp in copies:
                cp.wait()

            def row_body(i, _):
                for k in range(2):
                    r = i * 2 + k
                    rb = r * F
                    sa = [None] * 4
                    qa = [None] * 4
                    for f in range(F):
                        e = ebuf[rb + f]
                        a = f % 4
                        sa[a] = e if sa[a] is None else sa[a] + e
                        qa[a] = e * e if qa[a] is None else qa[a] + e * e
                    s = (sa[0] + sa[1]) + (sa[2] + sa[3])
                    ss = (qa[0] + qa[1]) + (qa[2] + qa[3])
                    v1 = plsc.load_gather(fcbuf, [rb + iota])
                    i2 = jnp.minimum(rb + L + iota, ppc - 1)
                    v2 = plsc.load_gather(fcbuf, [i2]) * m_tail
                    zv = v1 + v2 + 0.5 * (s * s - ss) + bias_v
                    zc = jnp.cumsum(zv)
                    pos = zero_i + (c * CHUNK + r)
                    plsc.store_scatter(obuf, [pos], zc, mask=m_last)
                return 0

            lax.fori_loop(0, CHUNK // 2, row_body, 0)
            return 0

        lax.fori_loop(0, nch, chunk_body, 0)

        def sig_body(g, _):
            off = pl.multiple_of(g * L, L)
            v = obuf[pl.ds(off, L)]
            obuf[pl.ds(off, L)] = 1.0 / (1.0 + jnp.exp(-v))
            return 0

        lax.fori_loop(0, rpw // L, sig_body, 0)
        pltpu.sync_copy(obuf, out_hbm.at[pl.ds(w * rpw, rpw)])

    return fm(emb_table, fc_flat, x_flat, bias16)


def kernel(x, emb_table, fc_table, bias):
    B, F = x.shape
    total, D = emb_table.shape
    assert D == L and B % (NW * CHUNK) == 0 and total % F == 0
    x_flat = x.astype(jnp.int32).reshape(-1)
    emb_pad = jnp.pad(emb_table, ((0, 0), (0, 128 - D))).reshape(-1, D)
    fc_pad = jnp.pad(fc_table, ((0, 0), (0, 127))).reshape(-1)
    bias16 = jnp.pad(bias.astype(jnp.float32), (0, L - 1))
    return _fm_call(x_flat, emb_pad, fc_pad, bias16, B, F, D, total)


# speedup vs baseline: 1.1793x; 1.1793x over previous
import functools

import jax
import jax.numpy as jnp
from jax import lax
from jax.experimental import pallas as pl
from jax.experimental.pallas import tpu as pltpu
from jax.experimental.pallas import tpu_sc as plsc

L = 16
NC, NS = 2, 16
NW = NC * NS
GR = 128


def _fm_call(x_t, emb_t, fc_flat, bias16, B, F, D, total):
    field_size = total // F
    rpw = B // NW
    nch = rpw // GR
    ppc = GR * F
    ng = ppc // L
    emb_words = emb_t.shape[0]

    mesh = plsc.VectorSubcoreMesh(
        core_axis_name="c", subcore_axis_name="s", num_cores=NC, num_subcores=NS)

    @functools.partial(
        pl.kernel,
        out_type=jax.ShapeDtypeStruct((B,), jnp.float32),
        mesh=mesh,
        scratch_types=[
            pltpu.VMEM((ppc,), jnp.int32),
            pltpu.VMEM((ppc,), jnp.int32),
            *[pltpu.VMEM((ppc,), jnp.float32) for _ in range(D)],
            pltpu.VMEM((ppc,), jnp.float32),
            pltpu.VMEM((rpw,), jnp.float32),
            pltpu.VMEM((L,), jnp.float32),
            pltpu.SemaphoreType.DMA,
        ],
    )
    def fm(emb_hbm, fc_hbm, x_hbm, b_hbm, out_hbm,
           xbuf, fidxbuf, *rest):
        ebufs = rest[:D]
        fcbuf, obuf, bbuf, sem = rest[D:]
        w = lax.axis_index("s") * NC + lax.axis_index("c")
        pltpu.sync_copy(b_hbm, bbuf)
        bias_v = bbuf[pl.ds(0, L)]

        def chunk_body(c, _):
            grp = w * nch + c
            pltpu.sync_copy(x_hbm.at[pl.ds(grp * ppc, ppc)], xbuf)

            def idx_body(g, _):
                off = pl.multiple_of(g * L, L)
                xv = xbuf[pl.ds(off, L)]
                row = xv + (g >> 3) * field_size
                fidxbuf[pl.ds(off, L)] = row
                return 0

            lax.fori_loop(0, ng, idx_body, 0)

            copies = [pltpu.async_copy(fc_hbm.at[fidxbuf], fcbuf, sem)]
            for d in range(D):
                copies.append(pltpu.async_copy(
                    emb_hbm.at[pl.ds(d * total, total)].at[fidxbuf],
                    ebufs[d], sem))
            for cp in copies:
                cp.wait()

            def rows_body(i, _):
                rr = i * L
                lin = bias_v
                for f in range(F):
                    lin = lin + fcbuf[pl.ds(f * GR + rr, L)]
                zacc = lin
                for d in range(D):
                    s = None
                    ss = None
                    for f in range(F):
                        e = ebufs[d][pl.ds(f * GR + rr, L)]
                        s = e if s is None else s + e
                        ss = e * e if ss is None else ss + e * e
                    zacc = zacc + 0.5 * (s * s - ss)
                obuf[pl.ds(c * GR + rr, L)] = 1.0 / (1.0 + jnp.exp(-zacc))
                return 0

            lax.fori_loop(0, GR // L, rows_body, 0)
            return 0

        lax.fori_loop(0, nch, chunk_body, 0)
        pltpu.sync_copy(obuf, out_hbm.at[pl.ds(w * rpw, rpw)])

    return fm(emb_t, fc_flat, x_t, bias16)


def kernel(x, emb_table, fc_table, bias):
    B, F = x.shape
    total, D = emb_table.shape
    assert D == L and B % (NW * GR) == 0 and total % F == 0
    x_t = jnp.transpose(
        x.astype(jnp.int32).reshape(B // GR, GR, F), (0, 2, 1)).reshape(-1)
    emb_t = jnp.transpose(emb_table).reshape(-1)
    fc_flat = fc_table.reshape(-1)
    bias16 = jnp.broadcast_to(bias.astype(jnp.float32), (L,))
    return _fm_call(x_t, emb_t, fc_flat, bias16, B, F, D, total)


# speedup vs baseline: 3.8576x; 3.2710x over previous
import functools

import jax
import jax.numpy as jnp
from jax import lax
from jax.experimental import pallas as pl
from jax.experimental.pallas import tpu as pltpu
from jax.experimental.pallas import tpu_sc as plsc

L = 16
NC, NS = 2, 16
NW = NC * NS
GR = 128


def _fm_call(x_t, emb_cols, fc_flat, bias16, B, F, D, total):
    field_size = total // F
    rpw = B // NW
    nch = rpw // GR
    ppc = GR * F
    ng = ppc // L

    mesh = plsc.VectorSubcoreMesh(
        core_axis_name="c", subcore_axis_name="s", num_cores=NC, num_subcores=NS)

    @functools.partial(
        pl.kernel,
        out_type=jax.ShapeDtypeStruct((B,), jnp.float32),
        mesh=mesh,
        scratch_types=[
            pltpu.VMEM((ppc,), jnp.int32),
            pltpu.VMEM((ppc,), jnp.int32),
            *[pltpu.VMEM((ppc,), jnp.float32) for _ in range(D)],
            pltpu.VMEM((ppc,), jnp.float32),
            pltpu.VMEM((rpw,), jnp.float32),
            pltpu.VMEM((L,), jnp.float32),
            pltpu.SemaphoreType.DMA,
        ],
    )
    def fm(*args):
        cols_hbm = args[:D]
        fc_hbm, x_hbm, b_hbm, out_hbm, xbuf, fidxbuf = args[D:D + 6]
        ebufs = args[D + 6:2 * D + 6]
        fcbuf, obuf, bbuf, sem = args[2 * D + 6:]
        w = lax.axis_index("s") * NC + lax.axis_index("c")
        pltpu.sync_copy(b_hbm, bbuf)
        bias_v = bbuf[pl.ds(0, L)]

        def chunk_body(c, _):
            grp = w * nch + c
            pltpu.sync_copy(x_hbm.at[pl.ds(grp * ppc, ppc)], xbuf)

            def idx_body(g, _):
                off = pl.multiple_of(g * L, L)
                xv = xbuf[pl.ds(off, L)]
                row = xv + (g >> 3) * field_size
                fidxbuf[pl.ds(off, L)] = row
                return 0

            lax.fori_loop(0, ng, idx_body, 0)

            copies = [pltpu.async_copy(fc_hbm.at[fidxbuf], fcbuf, sem)]
            for d in range(D):
                copies.append(pltpu.async_copy(
                    cols_hbm[d].at[fidxbuf], ebufs[d], sem))
            for cp in copies:
                cp.wait()

            def rows_body(i, _):
                rr = i * L
                lin = bias_v
                for f in range(F):
                    lin = lin + fcbuf[pl.ds(f * GR + rr, L)]
                zacc = lin
                for d in range(D):
                    s = None
                    ss = None
                    for f in range(F):
                        e = ebufs[d][pl.ds(f * GR + rr, L)]
                        s = e if s is None else s + e
                        ss = e * e if ss is None else ss + e * e
                    zacc = zacc + 0.5 * (s * s - ss)
                obuf[pl.ds(c * GR + rr, L)] = 1.0 / (1.0 + jnp.exp(-zacc))
                return 0

            lax.fori_loop(0, GR // L, rows_body, 0)
            return 0

        lax.fori_loop(0, nch, chunk_body, 0)
        pltpu.sync_copy(obuf, out_hbm.at[pl.ds(w * rpw, rpw)])

    return fm(*emb_cols, fc_flat, x_t, bias16)


def kernel(x, emb_table, fc_table, bias):
    B, F = x.shape
    total, D = emb_table.shape
    assert D == L and B % (NW * GR) == 0 and total % F == 0
    x_t = jnp.transpose(
        x.astype(jnp.int32).reshape(B // GR, GR, F), (0, 2, 1)).reshape(-1)
    emb_cols = [emb_table[:, d] for d in range(D)]
    fc_flat = fc_table.reshape(-1)
    bias16 = jnp.broadcast_to(bias.astype(jnp.float32), (L,))
    return _fm_call(x_t, emb_cols, fc_flat, bias16, B, F, D, total)
